# baseline (speedup 1.0000x reference)
import functools

import jax
import jax.numpy as jnp
from jax import lax
from jax.experimental import pallas as pl
from jax.experimental.pallas import tpu as pltpu
from jax.experimental.pallas import tpu_sc as plsc

VOCAB = 1000000
EMB = 64
MAXLEN = 200
BATCH = 4096

NUM_CORES = 2
NUM_SUBCORES = 16
NW = NUM_CORES * NUM_SUBCORES
ROWS_PER_W = BATCH // NW
IDS_PER_W = ROWS_PER_W * MAXLEN
NB = 4
LOOKAHEAD = 2


def _make_kernel():
    mesh = plsc.VectorSubcoreMesh(core_axis_name="c", subcore_axis_name="s")

    @functools.partial(
        pl.kernel,
        mesh=mesh,
        out_type=jax.ShapeDtypeStruct((BATCH * MAXLEN, EMB), jnp.float32),
        scratch_types=[
            pltpu.VMEM((IDS_PER_W,), jnp.int32),
            pltpu.VMEM((NB, MAXLEN, EMB), jnp.float32),
            pltpu.VMEM((MAXLEN, EMB), jnp.float32),
            pltpu.SemaphoreType.DMA((NB,)),
            pltpu.SemaphoreType.DMA((NB,)),
        ],
        compiler_params=pltpu.CompilerParams(use_tc_tiling_on_sc=False),
    )
    def emb_kernel(x_hbm, tok_hbm, pos_hbm, out_hbm, idx_v, rows_v, pos_v,
                   gsem, osem):
        wid = lax.axis_index("s") * NUM_CORES + lax.axis_index("c")
        base_id = pl.multiple_of(wid * IDS_PER_W, 8)
        pltpu.sync_copy(pos_hbm, pos_v)
        pltpu.sync_copy(x_hbm.at[pl.ds(base_id, IDS_PER_W)], idx_v)

        def gather_descs(r):
            b = lax.rem(r, NB)
            off = r * MAXLEN
            return (
                (tok_hbm.at[idx_v.at[pl.ds(off, 128)]],
                 rows_v.at[b, pl.ds(0, 128)], gsem.at[b]),
                (tok_hbm.at[idx_v.at[pl.ds(off + 128, MAXLEN - 128)]],
                 rows_v.at[b, pl.ds(128, MAXLEN - 128)], gsem.at[b]),
            )

        def out_desc(r):
            b = lax.rem(r, NB)
            start = pl.multiple_of(base_id + r * MAXLEN, 8)
            return (rows_v.at[b], out_hbm.at[pl.ds(start, MAXLEN)], osem.at[b])

        def issue_gather(r):
            for d in gather_descs(r):
                pltpu.async_copy(*d)

        def wait_gather(r):
            for d in gather_descs(r):
                pltpu.make_async_copy(*d).wait()

        for r0 in range(LOOKAHEAD):
            issue_gather(r0)

        def row_body(r, carry):
            @pl.when(r + LOOKAHEAD < ROWS_PER_W)
            def _issue_ahead():
                @pl.when(r >= LOOKAHEAD)
                def _drain_old():
                    pltpu.make_async_copy(*out_desc(r - LOOKAHEAD)).wait()
                issue_gather(r + LOOKAHEAD)

            wait_gather(r)
            b = lax.rem(r, NB)

            def add_body(i, c2):
                for c in range(EMB // 16):
                    sl = pl.ds(c * 16, 16)
                    rows_v[b, i, sl] = rows_v[b, i, sl] + pos_v[i, sl]
                return c2

            lax.fori_loop(0, MAXLEN, add_body, 0)
            pltpu.async_copy(*out_desc(r))
            return carry

        lax.fori_loop(0, ROWS_PER_W, row_body, 0)
        for rr in range(ROWS_PER_W - NB, ROWS_PER_W):
            pltpu.make_async_copy(*out_desc(rr)).wait()

    return emb_kernel


_EMB_KERNEL = _make_kernel()


def kernel(x, token_table, pos_table):
    x_flat = x.reshape(-1).astype(jnp.int32)
    out = _EMB_KERNEL(x_flat, token_table, pos_table)
    return out.reshape(BATCH, MAXLEN, EMB)
